# baseline (speedup 1.0000x reference)
import functools

import jax
import jax.numpy as jnp
from jax.experimental import pallas as pl
from jax.experimental.pallas import tpu as pltpu


def _fused_body(idx_ref, xt_ref, xb_ref, w_ref, b_ref, out_ref, g_ref):
    k = idx_ref.shape[1]
    nb, bm, fout = out_ref.shape
    fin = w_ref.shape[0] // k

    def gather_group(ib, carry):
        base = ib * 8
        for j in range(1, k):
            rows = [xt_ref[pl.ds(idx_ref[base + r, j], 1), :] for r in range(8)]
            g_ref[j - 1, pl.ds(base, 8), :] = jnp.concatenate(rows, axis=0)
        return carry

    jax.lax.fori_loop(0, bm // 8, gather_group, 0, unroll=4)

    for n in range(nb):
        acc = jnp.dot(xb_ref[:, n * fin:(n + 1) * fin], w_ref[0:fin, :],
                      preferred_element_type=jnp.float32)
        for j in range(1, k):
            acc = acc + jnp.dot(g_ref[j - 1, :, n * fin:(n + 1) * fin],
                                w_ref[j * fin:(j + 1) * fin, :],
                                preferred_element_type=jnp.float32)
        acc = acc + b_ref[...]
        out_ref[n] = jnp.where(acc > 0, acc, jnp.exp(acc) - 1.0)


@jax.jit
def kernel(x, index_list, W, b):
    n, m, fin = x.shape
    kf, fout = W.shape
    k = index_list.shape[1]
    bm = 200
    nf = n * fin

    mp = ((m + 1 + 7) // 8) * 8
    xt = jnp.pad(x.transpose(1, 0, 2).reshape(m, nf), ((0, mp - m), (0, 0)))
    b2 = b.reshape(1, fout)

    out = pl.pallas_call(
        _fused_body,
        grid=(m // bm,),
        in_specs=[
            pl.BlockSpec((bm, k), lambda j: (j, 0), memory_space=pltpu.SMEM),
            pl.BlockSpec((mp, nf), lambda j: (0, 0)),
            pl.BlockSpec((bm, nf), lambda j: (j, 0)),
            pl.BlockSpec((kf, fout), lambda j: (0, 0)),
            pl.BlockSpec((1, fout), lambda j: (0, 0)),
        ],
        out_specs=pl.BlockSpec((n, bm, fout), lambda j: (0, j, 0)),
        out_shape=jax.ShapeDtypeStruct((n, m, fout), jnp.float32),
        scratch_shapes=[pltpu.VMEM((k - 1, bm, nf), jnp.float32)],
        compiler_params=pltpu.CompilerParams(
            dimension_semantics=("parallel",)),
    )(index_list, xt, xt, W, b2)
    return out


# speedup vs baseline: 1.0729x; 1.0729x over previous
import functools

import jax
import jax.numpy as jnp
from jax.experimental import pallas as pl
from jax.experimental.pallas import tpu as pltpu


def _fused_body(idx_ref, xt_ref, xb_ref, w_ref, b_ref, out_ref, g_ref):
    k = idx_ref.shape[1]
    nb, bm, fout = out_ref.shape
    fin = w_ref.shape[0] // k

    def gather_group(ib, carry):
        base = ib * 8
        for j in range(1, k):
            rows = [xt_ref[pl.ds(idx_ref[base + r, j], 1), :] for r in range(8)]
            g_ref[j - 1, pl.ds(base, 8), :] = jnp.concatenate(rows, axis=0)
        return carry

    jax.lax.fori_loop(0, bm // 8, gather_group, 0, unroll=5)

    for n in range(nb):
        acc = jnp.dot(xb_ref[:, n * fin:(n + 1) * fin], w_ref[0:fin, :],
                      preferred_element_type=jnp.float32)
        for j in range(1, k):
            acc = acc + jnp.dot(g_ref[j - 1, :, n * fin:(n + 1) * fin],
                                w_ref[j * fin:(j + 1) * fin, :],
                                preferred_element_type=jnp.float32)
        acc = acc + b_ref[...]
        out_ref[n] = jnp.where(acc > 0, acc, jnp.exp(acc) - 1.0)


@jax.jit
def kernel(x, index_list, W, b):
    n, m, fin = x.shape
    kf, fout = W.shape
    k = index_list.shape[1]
    bm = 400
    nf = n * fin

    mp = ((m + 1 + 7) // 8) * 8
    xt = jnp.pad(x.transpose(1, 0, 2).reshape(m, nf), ((0, mp - m), (0, 0)))
    b2 = b.reshape(1, fout)

    out = pl.pallas_call(
        _fused_body,
        grid=(m // bm,),
        in_specs=[
            pl.BlockSpec((bm, k), lambda j: (j, 0), memory_space=pltpu.SMEM),
            pl.BlockSpec((mp, nf), lambda j: (0, 0)),
            pl.BlockSpec((bm, nf), lambda j: (j, 0)),
            pl.BlockSpec((kf, fout), lambda j: (0, 0)),
            pl.BlockSpec((1, fout), lambda j: (0, 0)),
        ],
        out_specs=pl.BlockSpec((n, bm, fout), lambda j: (0, j, 0)),
        out_shape=jax.ShapeDtypeStruct((n, m, fout), jnp.float32),
        scratch_shapes=[pltpu.VMEM((k - 1, bm, nf), jnp.float32)],
        compiler_params=pltpu.CompilerParams(
            dimension_semantics=("parallel",)),
    )(index_list, xt, xt, W, b2)
    return out


# speedup vs baseline: 1.1237x; 1.0474x over previous
import functools

import jax
import jax.numpy as jnp
from jax.experimental import pallas as pl
from jax.experimental.pallas import tpu as pltpu


def _fused_body(idx_ref, xt_ref, xb_ref, w_ref, b_ref, out_ref, g_ref):
    k = idx_ref.shape[1]
    nb, bm, fout = out_ref.shape
    fin = w_ref.shape[0] // k

    def gather_group(ib, carry):
        base = ib * 8
        for j in range(1, k):
            rows = [xt_ref[pl.ds(idx_ref[base + r, j], 1), :] for r in range(8)]
            g_ref[j - 1, pl.ds(base, 8), :] = jnp.concatenate(rows, axis=0)
        return carry

    jax.lax.fori_loop(0, bm // 8, gather_group, 0, unroll=4)

    for n in range(nb):
        acc = jnp.dot(xb_ref[:, n * fin:(n + 1) * fin], w_ref[0:fin, :],
                      preferred_element_type=jnp.float32)
        for j in range(1, k):
            acc = acc + jnp.dot(g_ref[j - 1, :, n * fin:(n + 1) * fin],
                                w_ref[j * fin:(j + 1) * fin, :],
                                preferred_element_type=jnp.float32)
        acc = acc + b_ref[...]
        out_ref[n] = jnp.where(acc > 0, acc, jnp.exp(acc) - 1.0)


@jax.jit
def kernel(x, index_list, W, b):
    n, m, fin = x.shape
    kf, fout = W.shape
    k = index_list.shape[1]
    bm = 400
    nf = n * fin

    mp = ((m + 1 + 7) // 8) * 8
    xt = jnp.pad(x.transpose(1, 0, 2).reshape(m, nf), ((0, mp - m), (0, 0)))
    b2 = b.reshape(1, fout)

    out = pl.pallas_call(
        _fused_body,
        grid=(m // bm,),
        in_specs=[
            pl.BlockSpec((bm, k), lambda j: (j, 0), memory_space=pltpu.SMEM),
            pl.BlockSpec((mp, nf), lambda j: (0, 0)),
            pl.BlockSpec((bm, nf), lambda j: (j, 0)),
            pl.BlockSpec((kf, fout), lambda j: (0, 0)),
            pl.BlockSpec((1, fout), lambda j: (0, 0)),
        ],
        out_specs=pl.BlockSpec((n, bm, fout), lambda j: (0, j, 0)),
        out_shape=jax.ShapeDtypeStruct((n, m, fout), jnp.float32),
        scratch_shapes=[pltpu.VMEM((k - 1, bm, nf), jnp.float32)],
        compiler_params=pltpu.CompilerParams(
            dimension_semantics=("parallel",)),
    )(index_list, xt, xt, W, b2)
    return out
